# baseline (speedup 1.0000x reference)
import math

import jax
import jax.numpy as jnp
from jax.experimental import pallas as pl

_BATCH = 16
_SEQ = 24
_N = 512
_D = 128
_FEAT = 8
_TAU = 3
_K = 8
_BIG = 3.0e38


def _dotT(a, b):
    return jax.lax.dot_general(a, b, (((1,), (1,)), ((), ())),
                               preferred_element_type=jnp.float32)


def _dot(a, b):
    return jax.lax.dot_general(a, b, (((1,), (0,)), ((), ())),
                               preferred_element_type=jnp.float32)


def _body(hist_ref, xo_ref, wrow_ref,
          wq_ref, wk_ref, wv_ref, w1_ref, w2_ref,
          bq_ref, bv_ref, b1_ref, b2_ref,
          out_ref):
    wc = xo_ref[0, 0]
    wr = wrow_ref[0]
    wxc = jax.lax.broadcast_in_dim(wc[:, 4:5], (_N, _N), (0, 1))
    wyc = jax.lax.broadcast_in_dim(wc[:, 5:6], (_N, _N), (0, 1))
    wxr = jax.lax.broadcast_in_dim(wr[0:1, :], (_N, _N), (0, 1))
    wyr = jax.lax.broadcast_in_dim(wr[1:2, :], (_N, _N), (0, 1))
    dx = wxc - wxr
    dy = wyc - wyr
    d2 = dx * dx + dy * dy

    cur = d2
    for _ in range(_K):
        rmin = jnp.min(cur, axis=1, keepdims=True)
        cur = jnp.where(cur == rmin, _BIG, cur)
    mask = cur > d2

    hist = hist_ref[0].reshape(_TAU * _N, _D)
    he_last = hist[(_TAU - 1) * _N:, :]
    q = _dotT(he_last, wq_ref[...]) + bq_ref[...]
    qk = _dot(q, wk_ref[...]).astype(jnp.bfloat16)

    inv = 1.0 / math.sqrt(_D)
    hist_b = hist.astype(jnp.bfloat16)
    s = _dotT(qk, hist_b)
    e = jnp.concatenate(
        [jnp.where(mask, jnp.exp(s[:, i * _N:(i + 1) * _N] * inv), 0.0)
         for i in range(_TAU)], axis=1)
    den = jnp.sum(e, axis=1, keepdims=True)
    ctx = _dot(e.astype(jnp.bfloat16), hist_b) / den

    ctx = _dotT(ctx, wv_ref[...]) + bv_ref[...]
    h1 = _dotT(ctx, w1_ref[...]) + b1_ref[...]
    g = 0.5 * h1 * (1.0 + jnp.tanh(0.7978845608028654 *
                                   (h1 + 0.044715 * h1 * h1 * h1)))
    out_ref[0] = _dotT(g, w2_ref[...]) + b2_ref[...]


@jax.jit
def kernel(h_e, x_orig, Wq, bq, Wk, bk, Wv, bv, W1, b1, W2, b2):
    b, seq_len, n, d = h_e.shape
    t0 = seq_len - 1
    t_start = t0 - _TAU + 1
    assert t_start % _TAU == 0
    tb = t_start // _TAU

    last_wind = x_orig[t0, :, :, 4:6]
    wrow = jnp.pad(jnp.transpose(last_wind, (0, 2, 1)), ((0, 0), (0, 6), (0, 0)))

    full = lambda shape: pl.BlockSpec(shape, lambda i: (0,) * len(shape))

    out = pl.pallas_call(
        _body,
        grid=(b,),
        in_specs=[
            pl.BlockSpec((1, _TAU, n, d), lambda i: (i, tb, 0, 0)),
            pl.BlockSpec((1, 1, n, _FEAT), lambda i: (t0, i, 0, 0)),
            pl.BlockSpec((1, 8, n), lambda i: (i, 0, 0)),
            full((d, d)), full((d, d)), full((d, d)), full((d, d)), full((d, d)),
            full((1, d)), full((1, d)), full((1, d)), full((1, d)),
        ],
        out_specs=pl.BlockSpec((1, n, d), lambda i: (i, 0, 0)),
        out_shape=jax.ShapeDtypeStruct((b, n, d), jnp.float32),
    )(h_e, x_orig, wrow, Wq, Wk, Wv, W1, W2,
      bq.reshape(1, d), bv.reshape(1, d), b1.reshape(1, d), b2.reshape(1, d))
    return out


# speedup vs baseline: 1.4855x; 1.4855x over previous
import math

import jax
import jax.numpy as jnp
from jax.experimental import pallas as pl

_BATCH = 16
_SEQ = 24
_N = 512
_D = 128
_FEAT = 8
_TAU = 3
_K = 8
_BIG = 3.0e38


def _dotT(a, b):
    return jax.lax.dot_general(a, b, (((1,), (1,)), ((), ())),
                               preferred_element_type=jnp.float32)


def _dot(a, b):
    return jax.lax.dot_general(a, b, (((1,), (0,)), ((), ())),
                               preferred_element_type=jnp.float32)


def _body(hist_ref, wcol_ref, wrow_ref,
          wq_ref, wk_ref, wv_ref, w1_ref, w2_ref,
          bq_ref, bv_ref, b1_ref, b2_ref,
          out_ref):
    wc = wcol_ref[0]
    wr = wrow_ref[0]
    wxc = jax.lax.broadcast_in_dim(wc[:, 0:1], (_N, _N), (0, 1))
    wyc = jax.lax.broadcast_in_dim(wc[:, 1:2], (_N, _N), (0, 1))
    wxr = jax.lax.broadcast_in_dim(wr[0:1, :], (_N, _N), (0, 1))
    wyr = jax.lax.broadcast_in_dim(wr[1:2, :], (_N, _N), (0, 1))
    dx = wxc - wxr
    dy = wyc - wyr
    d2 = dx * dx + dy * dy

    cur = d2
    for _ in range(_K):
        rmin = jnp.min(cur, axis=1, keepdims=True)
        cur = jnp.where(cur == rmin, _BIG, cur)
    mask = cur > d2

    hist = hist_ref[0]
    he_last = hist[(_TAU - 1) * _N:, :]
    q = _dotT(he_last, wq_ref[...]) + bq_ref[...]
    qk = _dot(q, wk_ref[...]).astype(jnp.bfloat16)

    inv = 1.0 / math.sqrt(_D)
    hist_b = hist.astype(jnp.bfloat16)
    s = _dotT(qk, hist_b)
    e = jnp.concatenate(
        [jnp.where(mask, jnp.exp(s[:, i * _N:(i + 1) * _N] * inv), 0.0)
         for i in range(_TAU)], axis=1)
    den = jnp.sum(e, axis=1, keepdims=True)
    ctx = _dot(e.astype(jnp.bfloat16), hist_b) / den

    ctx = _dotT(ctx, wv_ref[...]) + bv_ref[...]
    h1 = _dotT(ctx, w1_ref[...]) + b1_ref[...]
    g = 0.5 * h1 * (1.0 + jnp.tanh(0.7978845608028654 *
                                   (h1 + 0.044715 * h1 * h1 * h1)))
    out_ref[0] = _dotT(g, w2_ref[...]) + b2_ref[...]


@jax.jit
def kernel(h_e, x_orig, Wq, bq, Wk, bk, Wv, bv, W1, b1, W2, b2):
    b, seq_len, n, d = h_e.shape
    t0 = seq_len - 1
    t_start = t0 - _TAU + 1
    hist = h_e[:, t_start:t0 + 1].reshape(b, _TAU * n, d)

    last_wind = x_orig[t0, :, :, 4:6]
    wcol = jnp.pad(last_wind, ((0, 0), (0, 0), (0, _FEAT - 2)))
    wrow = jnp.pad(jnp.transpose(last_wind, (0, 2, 1)), ((0, 0), (0, 6), (0, 0)))

    full = lambda shape: pl.BlockSpec(shape, lambda i: (0,) * len(shape))

    out = pl.pallas_call(
        _body,
        grid=(b,),
        in_specs=[
            pl.BlockSpec((1, _TAU * n, d), lambda i: (i, 0, 0)),
            pl.BlockSpec((1, n, _FEAT), lambda i: (i, 0, 0)),
            pl.BlockSpec((1, 8, n), lambda i: (i, 0, 0)),
            full((d, d)), full((d, d)), full((d, d)), full((d, d)), full((d, d)),
            full((1, d)), full((1, d)), full((1, d)), full((1, d)),
        ],
        out_specs=pl.BlockSpec((1, n, d), lambda i: (i, 0, 0)),
        out_shape=jax.ShapeDtypeStruct((b, n, d), jnp.float32),
    )(hist, wcol, wrow, Wq, Wk, Wv, W1, W2,
      bq.reshape(1, d), bv.reshape(1, d), b1.reshape(1, d), b2.reshape(1, d))
    return out


# speedup vs baseline: 1.7665x; 1.1892x over previous
import math

import jax
import jax.numpy as jnp
from jax.experimental import pallas as pl

_BATCH = 16
_SEQ = 24
_N = 512
_D = 128
_FEAT = 8
_TAU = 3
_K = 8
_BIG = 3.0e38


def _dotT(a, b):
    return jax.lax.dot_general(a, b, (((1,), (1,)), ((), ())),
                               preferred_element_type=jnp.float32)


def _dot(a, b):
    return jax.lax.dot_general(a, b, (((1,), (0,)), ((), ())),
                               preferred_element_type=jnp.float32)


def _body(h0_ref, h1_ref, h2_ref, wcol_ref, wrow_ref,
          wq_ref, wk_ref, wv_ref, w1_ref, w2_ref,
          bq_ref, bv_ref, b1_ref, b2_ref,
          out_ref):
    wc = wcol_ref[0]
    wr = wrow_ref[0]
    wxc = jax.lax.broadcast_in_dim(wc[:, 0:1], (_N, _N), (0, 1))
    wyc = jax.lax.broadcast_in_dim(wc[:, 1:2], (_N, _N), (0, 1))
    wxr = jax.lax.broadcast_in_dim(wr[0:1, :], (_N, _N), (0, 1))
    wyr = jax.lax.broadcast_in_dim(wr[1:2, :], (_N, _N), (0, 1))
    dx = wxc - wxr
    dy = wyc - wyr
    d2 = dx * dx + dy * dy

    cur = d2
    for _ in range(_K):
        rmin = jnp.min(cur, axis=1, keepdims=True)
        cur = jnp.where(cur == rmin, _BIG, cur)
    mask = cur > d2

    he_last = h2_ref[0, 0]
    q = _dotT(he_last, wq_ref[...]) + bq_ref[...]
    qk = _dot(q, wk_ref[...]).astype(jnp.bfloat16)

    inv = 1.0 / math.sqrt(_D)
    den = jnp.zeros((_N, 1), jnp.float32)
    ctx = jnp.zeros((_N, _D), jnp.float32)
    for h_ref in (h0_ref, h1_ref, h2_ref):
        h_t = h_ref[0, 0].astype(jnp.bfloat16)
        s_t = _dotT(qk, h_t)
        e_t = jnp.where(mask, jnp.exp(s_t * inv), 0.0)
        den = den + jnp.sum(e_t, axis=1, keepdims=True)
        ctx = ctx + _dot(e_t.astype(jnp.bfloat16), h_t)
    ctx = ctx / den

    ctx = _dotT(ctx, wv_ref[...]) + bv_ref[...]
    h1 = _dotT(ctx, w1_ref[...]) + b1_ref[...]
    g = 0.5 * h1 * (1.0 + jnp.tanh(0.7978845608028654 *
                                   (h1 + 0.044715 * h1 * h1 * h1)))
    out_ref[0] = _dotT(g, w2_ref[...]) + b2_ref[...]


@jax.jit
def kernel(h_e, x_orig, Wq, bq, Wk, bk, Wv, bv, W1, b1, W2, b2):
    b, seq_len, n, d = h_e.shape
    t0 = seq_len - 1
    t_start = t0 - _TAU + 1

    last_wind = x_orig[t0, :, :, 4:6]
    wcol = jnp.pad(last_wind, ((0, 0), (0, 0), (0, _FEAT - 2)))
    wrow = jnp.pad(jnp.transpose(last_wind, (0, 2, 1)), ((0, 0), (0, 6), (0, 0)))

    full = lambda shape: pl.BlockSpec(shape, lambda i: (0,) * len(shape))
    h_t_spec = lambda t: pl.BlockSpec((1, 1, n, d), lambda i, t=t: (i, t, 0, 0))

    out = pl.pallas_call(
        _body,
        grid=(b,),
        in_specs=[
            h_t_spec(t_start), h_t_spec(t_start + 1), h_t_spec(t0),
            pl.BlockSpec((1, n, _FEAT), lambda i: (i, 0, 0)),
            pl.BlockSpec((1, 8, n), lambda i: (i, 0, 0)),
            full((d, d)), full((d, d)), full((d, d)), full((d, d)), full((d, d)),
            full((1, d)), full((1, d)), full((1, d)), full((1, d)),
        ],
        out_specs=pl.BlockSpec((1, n, d), lambda i: (i, 0, 0)),
        out_shape=jax.ShapeDtypeStruct((b, n, d), jnp.float32),
    )(h_e, h_e, h_e, wcol, wrow, Wq, Wk, Wv, W1, W2,
      bq.reshape(1, d), bv.reshape(1, d), b1.reshape(1, d), b2.reshape(1, d))
    return out
